# Initial kernel scaffold; baseline (speedup 1.0000x reference)
import jax
import jax.numpy as jnp
from jax.experimental import pallas as pl


def kernel(x, edge_index, Wt, Ws, Wc, Wq, bq):
    raise NotImplementedError("write your pallas kernel here")


# speedup vs baseline: 17.1472x; 17.1472x over previous
import functools

import jax
import jax.numpy as jnp
from jax import lax
from jax.experimental import pallas as pl
from jax.experimental.pallas import tpu as pltpu
from jax.experimental.pallas import tpu_sc as plsc

N = 10000
E = 320000
D = 128
H = 4
DH = 32

NC = 2
NS = 16
NW = NC * NS
L = 16

EPW = E // NW
CH = 80
NCHUNK = EPW // CH
AD = 144
NPAD = 10240
RPS = NPAD // NS
ZR = 32

_mesh = plsc.VectorSubcoreMesh(
    core_axis_name="c", subcore_axis_name="s", num_cores=NC, num_subcores=NS)


def _proj_body(x_ref, wt_ref, ws_ref, wc_ref, q_ref, k_ref, v_ref):
    x = x_ref[...]
    dn = (((1,), (1,)), ((), ()))
    q_ref[...] = lax.dot_general(x, wt_ref[...], dn,
                                 preferred_element_type=jnp.float32)
    k_ref[...] = lax.dot_general(x, ws_ref[...], dn,
                                 preferred_element_type=jnp.float32)
    v_ref[...] = lax.dot_general(x, wc_ref[...], dn,
                                 preferred_element_type=jnp.float32)


def _project(x, Wt, Ws, Wc):
    B = 400
    return pl.pallas_call(
        _proj_body,
        grid=(N // B,),
        in_specs=[
            pl.BlockSpec((B, D), lambda i: (i, 0)),
            pl.BlockSpec((D, D), lambda i: (0, 0)),
            pl.BlockSpec((D, D), lambda i: (0, 0)),
            pl.BlockSpec((D, D), lambda i: (0, 0)),
        ],
        out_specs=[pl.BlockSpec((B, D), lambda i: (i, 0))] * 3,
        out_shape=[jax.ShapeDtypeStruct((N, D), jnp.float32)] * 3,
    )(x, Wt, Ws, Wc)


def _scores_body(q_hbm, k_hbm, src_hbm, dst_hbm, scores_hbm, maxes_hbm,
                 idx_s, idx_d, qrows, krows, scb, mbuf, sem):
    cid = lax.axis_index("c")
    sid = lax.axis_index("s")
    wid = sid * NC + cid
    base = wid * EPW
    lane = lax.iota(jnp.int32, L)

    def chunk(ci, mx):
        off = base + ci * CH
        pltpu.sync_copy(dst_hbm.at[pl.ds(off, CH)], idx_d)
        pltpu.sync_copy(src_hbm.at[pl.ds(off, CH)], idx_s)
        cp1 = pltpu.async_copy(q_hbm.at[idx_d], qrows, sem)
        cp2 = pltpu.async_copy(k_hbm.at[idx_s], krows, sem)
        cp1.wait()
        cp2.wait()

        def group(g, carry):
            rowv = g * L + lane
            for h in range(H):
                acc = jnp.zeros((L,), jnp.float32)
                for c in range(h * DH, (h + 1) * DH):
                    colv = jnp.full((L,), c, jnp.int32)
                    acc = acc + (plsc.load_gather(qrows, [rowv, colv]) *
                                 plsc.load_gather(krows, [rowv, colv]))
                scb[pl.ds(h * CH + g * L, L)] = acc
            return carry

        lax.fori_loop(0, CH // L, group, 0)
        pltpu.sync_copy(scb.at[pl.ds(0, CH * H)],
                        scores_hbm.at[pl.ds(off * H, CH * H)])
        for i in range(CH * H // L):
            mx = jnp.maximum(mx, scb[pl.ds(i * L, L)])
        return mx

    mx0 = jnp.full((L,), -1e30, jnp.float32)
    mx = lax.fori_loop(0, NCHUNK, chunk, mx0)
    mbuf[...] = mx
    pltpu.sync_copy(mbuf, maxes_hbm.at[pl.ds(pl.multiple_of(wid * L, 8), L)])


_scores_call = functools.partial(
    pl.kernel,
    _scores_body,
    out_type=[
        jax.ShapeDtypeStruct((E * H,), jnp.float32),
        jax.ShapeDtypeStruct((NW * L,), jnp.float32),
    ],
    mesh=_mesh,
    scratch_types=[
        pltpu.VMEM((CH,), jnp.int32),
        pltpu.VMEM((CH,), jnp.int32),
        pltpu.VMEM((CH, D), jnp.float32),
        pltpu.VMEM((CH, D), jnp.float32),
        pltpu.VMEM((CH * H + L,), jnp.float32),
        pltpu.VMEM((L,), jnp.float32),
        pltpu.SemaphoreType.DMA,
    ],
    compiler_params=pltpu.CompilerParams(needs_layout_passes=False),
)()


def _agg_body(v_hbm, src_hbm, dst_hbm, scores_hbm, maxes_hbm, agg_hbm,
              idx_s, idx_d, vrows, msg, scb, mbuf, zbuf, sem, accum):
    cid = lax.axis_index("c")
    sid = lax.axis_index("s")
    wid = sid * NC + cid
    base = wid * EPW

    pltpu.sync_copy(maxes_hbm, mbuf)
    m = mbuf[pl.ds(0, L)]
    for i in range(1, NW):
        m = jnp.maximum(m, mbuf[pl.ds(i * L, L)])
    C = m[0]
    for i in range(1, L):
        C = jnp.maximum(C, m[i])

    zero = jnp.zeros((L,), jnp.float32)
    for r in range(ZR):
        for j in range(AD // L):
            zbuf[r, pl.ds(j * L, L)] = zero
    row0 = pl.multiple_of(sid * RPS, 8)
    for r in range(RPS // ZR):
        pltpu.sync_copy(zbuf, accum.at[pl.ds(row0 + r * ZR, ZR)])
    plsc.subcore_barrier()

    lane = lax.iota(jnp.int32, L)

    def chunk(ci, carry):
        off = base + ci * CH
        pltpu.sync_copy(dst_hbm.at[pl.ds(off, CH)], idx_d)
        pltpu.sync_copy(src_hbm.at[pl.ds(off, CH)], idx_s)
        cp = pltpu.async_copy(v_hbm.at[idx_s], vrows, sem)
        pltpu.sync_copy(scores_hbm.at[pl.ds(off * H, CH * H)],
                        scb.at[pl.ds(0, CH * H)])
        for i in range(CH * H // L):
            scb[pl.ds(i * L, L)] = jnp.exp(scb[pl.ds(i * L, L)] - C)
        cp.wait()

        def group(g, carry2):
            exv = [scb[pl.ds(h * CH + g * L, L)] for h in range(H)]
            for t in range(L):
                e = g * L + t
                dv = jnp.zeros((L,), jnp.float32)
                for h in range(H):
                    mh = jnp.full((L,), exv[h][t], jnp.float32)
                    msg[e, pl.ds(h * DH, L)] = (
                        vrows[e, pl.ds(h * DH, L)] * mh)
                    msg[e, pl.ds(h * DH + L, L)] = (
                        vrows[e, pl.ds(h * DH + L, L)] * mh)
                    dv = jnp.where(lane == h, mh, dv)
                msg[e, pl.ds(D, L)] = dv
            return carry2

        lax.fori_loop(0, CH // L, group, 0)
        pltpu.sync_copy(msg, accum.at[idx_d], add=True)
        return carry

    lax.fori_loop(0, NCHUNK, chunk, 0)
    plsc.subcore_barrier()
    row0b = pl.multiple_of(sid * RPS, 8)
    pltpu.sync_copy(accum.at[pl.ds(row0b, RPS)],
                    agg_hbm.at[cid, pl.ds(row0b, RPS)])


_agg_call = functools.partial(
    pl.kernel,
    _agg_body,
    out_type=jax.ShapeDtypeStruct((NC, NPAD, AD), jnp.float32),
    mesh=_mesh,
    scratch_types=[
        pltpu.VMEM((CH,), jnp.int32),
        pltpu.VMEM((CH,), jnp.int32),
        pltpu.VMEM((CH, D), jnp.float32),
        pltpu.VMEM((CH, AD), jnp.float32),
        pltpu.VMEM((CH * H + L,), jnp.float32),
        pltpu.VMEM((NW * L,), jnp.float32),
        pltpu.VMEM((ZR, AD), jnp.float32),
        pltpu.SemaphoreType.DMA,
        pltpu.VMEM_SHARED((NPAD, AD), jnp.float32),
    ],
    compiler_params=pltpu.CompilerParams(
        needs_layout_passes=False, use_tc_tiling_on_sc=False),
)()


def _out_body(a0_ref, a1_ref, b1_ref, wq_ref, bq_ref, x_ref, o_ref):
    y = a0_ref[...] + a1_ref[...]
    den = lax.dot_general(y, b1_ref[...], (((1,), (0,)), ((), ())),
                          preferred_element_type=jnp.float32)
    num = y[:, :D]
    z = num / (den + 1e-16)
    z = lax.dot_general(z, wq_ref[...], (((1,), (1,)), ((), ())),
                        preferred_element_type=jnp.float32)
    o_ref[...] = jnp.maximum(z + bq_ref[...], 0.0) + x_ref[...]


def _finalize(agg, Wq, bq, x):
    B = 400
    b1 = jnp.zeros((AD, D), jnp.float32)
    heads = jnp.repeat(jnp.arange(H), DH)
    b1 = b1.at[D + heads, jnp.arange(D)].set(1.0)
    return pl.pallas_call(
        _out_body,
        grid=(N // B,),
        in_specs=[
            pl.BlockSpec((B, AD), lambda i: (i, 0)),
            pl.BlockSpec((B, AD), lambda i: (i, 0)),
            pl.BlockSpec((AD, D), lambda i: (0, 0)),
            pl.BlockSpec((D, D), lambda i: (0, 0)),
            pl.BlockSpec((1, D), lambda i: (0, 0)),
            pl.BlockSpec((B, D), lambda i: (i, 0)),
        ],
        out_specs=pl.BlockSpec((B, D), lambda i: (i, 0)),
        out_shape=jax.ShapeDtypeStruct((N, D), jnp.float32),
    )(agg[0], agg[1], b1, Wq, bq.reshape(1, D), x)


def kernel(x, edge_index, Wt, Ws, Wc, Wq, bq):
    src = edge_index[0].astype(jnp.int32)
    dst = edge_index[1].astype(jnp.int32)
    q, k, v = _project(x, Wt, Ws, Wc)
    scores, maxes = _scores_call(q, k, src, dst)
    agg = _agg_call(v, src, dst, scores, maxes)
    return _finalize(agg, Wq, bq, x)
